# baseline (speedup 1.0000x reference)
import functools

import jax
import jax.numpy as jnp
from jax import lax
from jax.experimental import pallas as pl
from jax.experimental.pallas import tpu as pltpu
from jax.experimental.pallas import tpu_sc as plsc

V = 1000000
D = 64
B = 16384

NC, NS, L = 2, 16, 16
NW = NC * NS
BPW = B // NW
CHUNK = 128
NCHUNK = BPW // CHUNK
NBLK = CHUNK // L
PW = D // 2


def _body(wi_i_hbm, wi_j_hbm, pk_i_hbm, pk_j_hbm, bb_hbm, out_hbm,
          widx_i, widx_j, qidx_i, qidx_j, bidx_i, bidx_j,
          rows_i, rows_j, brow_i, brow_j, out_v, sem):
    wid = lax.axis_index("s") * NC + lax.axis_index("c")
    base = wid * BPW

    pltpu.sync_copy(wi_i_hbm.at[pl.ds(base, BPW)], widx_i)
    pltpu.sync_copy(wi_j_hbm.at[pl.ds(base, BPW)], widx_j)

    def stage(t, carry):
        s = pl.ds(t * L, L)
        wv_i = widx_i[s]
        wv_j = widx_j[s]
        qidx_i[s] = wv_i >> 2
        qidx_j[s] = wv_j >> 2
        bidx_i[s] = wv_i >> 7
        bidx_j[s] = (wv_j + V) >> 7
        return carry

    lax.fori_loop(0, BPW // L, stage, 0, unroll=False)

    iota = lax.iota(jnp.int32, L)
    lo_mask = jnp.full((L,), 0xFFFF, jnp.int32) << 16

    def chunk(c, carry):
        c0 = c * CHUNK
        g1 = pltpu.async_copy(pk_i_hbm.at[qidx_i.at[pl.ds(c0, CHUNK)]], rows_i, sem)
        g2 = pltpu.async_copy(pk_j_hbm.at[qidx_j.at[pl.ds(c0, CHUNK)]], rows_j, sem)
        g3 = pltpu.async_copy(bb_hbm.at[bidx_i.at[pl.ds(c0, CHUNK)]], brow_i, sem)
        g4 = pltpu.async_copy(bb_hbm.at[bidx_j.at[pl.ds(c0, CHUNK)]], brow_j, sem)
        g1.wait()
        g2.wait()
        g3.wait()
        g4.wait()

        def block(b, carry2):
            g0 = c0 + b * L
            s = pl.ds(g0, L)
            lrvec = b * L + iota
            wv_i = widx_i[s]
            wv_j = widx_j[s]
            col_i = (wv_i & 3) * PW
            col_j = (wv_j & 3) * PW
            acc = plsc.load_gather(brow_i, [lrvec, wv_i & 127]) + \
                plsc.load_gather(brow_j, [lrvec, (wv_j + V) & 127])
            for p in range(PW):
                pi = plsc.load_gather(rows_i, [lrvec, col_i + p])
                pj = plsc.load_gather(rows_j, [lrvec, col_j + p])
                fi_lo = plsc.bitcast(pi << 16, jnp.float32)
                fj_lo = plsc.bitcast(pj << 16, jnp.float32)
                fi_hi = plsc.bitcast(pi & lo_mask, jnp.float32)
                fj_hi = plsc.bitcast(pj & lo_mask, jnp.float32)
                acc = acc + fi_lo * fj_lo + fi_hi * fj_hi
            out_v[s] = acc
            return carry2

        lax.fori_loop(0, NBLK, block, 0, unroll=False)
        return carry

    lax.fori_loop(0, NCHUNK, chunk, 0, unroll=False)

    pltpu.sync_copy(out_v, out_hbm.at[pl.ds(base, BPW)])


@functools.partial(jax.jit, static_argnames=())
def kernel(word_i, word_j, wi, wj, bi, bj):
    mesh = plsc.VectorSubcoreMesh(core_axis_name="c", subcore_axis_name="s")
    k = pl.kernel(
        _body,
        out_type=jax.ShapeDtypeStruct((B,), jnp.float32),
        mesh=mesh,
        compiler_params=pltpu.CompilerParams(needs_layout_passes=False),
        scratch_types=[
            pltpu.VMEM((BPW,), jnp.int32),
            pltpu.VMEM((BPW,), jnp.int32),
            pltpu.VMEM((BPW,), jnp.int32),
            pltpu.VMEM((BPW,), jnp.int32),
            pltpu.VMEM((BPW,), jnp.int32),
            pltpu.VMEM((BPW,), jnp.int32),
            pltpu.VMEM((CHUNK, 128), jnp.int32),
            pltpu.VMEM((CHUNK, 128), jnp.int32),
            pltpu.VMEM((CHUNK, 128), jnp.float32),
            pltpu.VMEM((CHUNK, 128), jnp.float32),
            pltpu.VMEM((BPW,), jnp.float32),
            pltpu.SemaphoreType.DMA,
        ],
    )
    word_i = word_i.astype(jnp.int32)
    word_j = word_j.astype(jnp.int32)
    pk_i = lax.bitcast_convert_type(
        wi.astype(jnp.bfloat16).reshape(V // 4, 128, 2), jnp.int32)
    pk_j = lax.bitcast_convert_type(
        wj.astype(jnp.bfloat16).reshape(V // 4, 128, 2), jnp.int32)
    bb = jnp.concatenate([bi.reshape(V), bj.reshape(V)]).reshape(2 * V // 128, 128)
    return k(word_i, word_j, pk_i, pk_j, bb)


# speedup vs baseline: 11.0925x; 11.0925x over previous
import functools

import jax
import jax.numpy as jnp
from jax import lax
from jax.experimental import pallas as pl
from jax.experimental.pallas import tpu as pltpu
from jax.experimental.pallas import tpu_sc as plsc

V = 1000000
D = 64
B = 16384

NC, NS, L = 2, 16, 16
NW = NC * NS
BPW = B // NW
CHUNK = 128
NCHUNK = BPW // CHUNK
NBLK = CHUNK // L
PW = D // 2


def _body(wi_i_hbm, wi_j_hbm, pk_i_hbm, pk_j_hbm, bb_hbm, out_hbm,
          widx_i, widx_j, qidx_i, qidx_j, bidx_i, bidx_j,
          rows_i, rows_j, brow_i, brow_j, out_v, sem):
    wid = lax.axis_index("s") * NC + lax.axis_index("c")
    base = wid * BPW

    pltpu.sync_copy(wi_i_hbm.at[pl.ds(base, BPW)], widx_i)
    pltpu.sync_copy(wi_j_hbm.at[pl.ds(base, BPW)], widx_j)

    def stage(t, carry):
        s = pl.ds(t * L, L)
        wv_i = widx_i[s]
        wv_j = widx_j[s]
        qidx_i[s] = wv_i >> 2
        qidx_j[s] = wv_j >> 2
        bidx_i[s] = wv_i >> 7
        bidx_j[s] = (wv_j + V) >> 7
        return carry

    lax.fori_loop(0, BPW // L, stage, 0, unroll=False)

    iota = lax.iota(jnp.int32, L)
    lo_mask = jnp.full((L,), 0xFFFF, jnp.int32) << 16

    def chunk(c, carry):
        c0 = c * CHUNK
        g1 = pltpu.async_copy(pk_i_hbm.at[qidx_i.at[pl.ds(c0, CHUNK)]], rows_i, sem)
        g2 = pltpu.async_copy(pk_j_hbm.at[qidx_j.at[pl.ds(c0, CHUNK)]], rows_j, sem)
        g3 = pltpu.async_copy(bb_hbm.at[bidx_i.at[pl.ds(c0, CHUNK)]], brow_i, sem)
        g4 = pltpu.async_copy(bb_hbm.at[bidx_j.at[pl.ds(c0, CHUNK)]], brow_j, sem)
        g1.wait()
        g2.wait()
        g3.wait()
        g4.wait()

        def block(b, carry2):
            g0 = c0 + b * L
            s = pl.ds(g0, L)
            lrvec = b * L + iota
            wv_i = widx_i[s]
            wv_j = widx_j[s]
            col_i = (wv_i & 3) * PW
            col_j = (wv_j & 3) * PW
            acc = plsc.load_gather(brow_i, [lrvec, wv_i & 127]) + \
                plsc.load_gather(brow_j, [lrvec, (wv_j + V) & 127])
            for p in range(PW):
                pi = plsc.load_gather(rows_i, [lrvec, col_i + p])
                pj = plsc.load_gather(rows_j, [lrvec, col_j + p])
                fi_lo = plsc.bitcast(pi << 16, jnp.float32)
                fj_lo = plsc.bitcast(pj << 16, jnp.float32)
                fi_hi = plsc.bitcast(pi & lo_mask, jnp.float32)
                fj_hi = plsc.bitcast(pj & lo_mask, jnp.float32)
                acc = acc + fi_lo * fj_lo + fi_hi * fj_hi
            out_v[s] = acc
            return carry2

        lax.fori_loop(0, NBLK, block, 0, unroll=False)
        return carry

    lax.fori_loop(0, NCHUNK, chunk, 0, unroll=False)

    pltpu.sync_copy(out_v, out_hbm.at[pl.ds(base, BPW)])


@functools.partial(jax.jit, static_argnames=())
def kernel(word_i, word_j, wi, wj, bi, bj):
    mesh = plsc.VectorSubcoreMesh(core_axis_name="c", subcore_axis_name="s")
    k = pl.kernel(
        _body,
        out_type=jax.ShapeDtypeStruct((B,), jnp.float32),
        mesh=mesh,
        compiler_params=pltpu.CompilerParams(needs_layout_passes=False),
        scratch_types=[
            pltpu.VMEM((BPW,), jnp.int32),
            pltpu.VMEM((BPW,), jnp.int32),
            pltpu.VMEM((BPW,), jnp.int32),
            pltpu.VMEM((BPW,), jnp.int32),
            pltpu.VMEM((BPW,), jnp.int32),
            pltpu.VMEM((BPW,), jnp.int32),
            pltpu.VMEM((CHUNK, 128), jnp.int32),
            pltpu.VMEM((CHUNK, 128), jnp.int32),
            pltpu.VMEM((CHUNK, 128), jnp.float32),
            pltpu.VMEM((CHUNK, 128), jnp.float32),
            pltpu.VMEM((BPW,), jnp.float32),
            pltpu.SemaphoreType.DMA,
        ],
    )
    word_i = word_i.astype(jnp.int32)
    word_j = word_j.astype(jnp.int32)

    def rne16(u):
        return (u + 0x7FFF + ((u >> 16) & 1)) >> 16

    def pack(w):
        u = lax.bitcast_convert_type(w, jnp.int32)
        lo = rne16(u[:, 0::2]) & 0xFFFF
        hi = rne16(u[:, 1::2]) << 16
        return (lo | hi).reshape(V // 4, 128)

    pk_i = pack(wi)
    pk_j = pack(wj)
    bb = jnp.concatenate([bi.reshape(V), bj.reshape(V)]).reshape(2 * V // 128, 128)
    return k(word_i, word_j, pk_i, pk_j, bb)


# speedup vs baseline: 29.9081x; 2.6963x over previous
import functools

import jax
import jax.numpy as jnp
from jax import lax
from jax.experimental import pallas as pl
from jax.experimental.pallas import tpu as pltpu
from jax.experimental.pallas import tpu_sc as plsc

V = 1000000
D = 64
B = 16384

NC, NS, L = 2, 16, 16
NW = NC * NS
BPW = B // NW
CHUNK = 128
NCHUNK = BPW // CHUNK
NBLK = CHUNK // L
PW = D // 2


def _body(wi_i_hbm, wi_j_hbm, pk_i_hbm, pk_j_hbm, bb_hbm, out_hbm,
          widx_i, widx_j, qidx_i, qidx_j, bidx_i, bidx_j,
          rows_i, rows_j, brow_i, brow_j, out_v, sem):
    wid = lax.axis_index("s") * NC + lax.axis_index("c")
    base = wid * BPW

    pltpu.sync_copy(wi_i_hbm.at[pl.ds(base, BPW)], widx_i)
    pltpu.sync_copy(wi_j_hbm.at[pl.ds(base, BPW)], widx_j)

    def stage(t, carry):
        s = pl.ds(t * L, L)
        wv_i = widx_i[s]
        wv_j = widx_j[s]
        qidx_i[s] = wv_i >> 2
        qidx_j[s] = wv_j >> 2
        bidx_i[s] = wv_i >> 7
        bidx_j[s] = (wv_j + V) >> 7
        return carry

    lax.fori_loop(0, BPW // L, stage, 0, unroll=False)

    iota = lax.iota(jnp.int32, L)
    lo_mask = jnp.full((L,), 0xFFFF, jnp.int32) << 16

    def chunk(c, carry):
        c0 = c * CHUNK
        g1 = pltpu.async_copy(pk_i_hbm.at[qidx_i.at[pl.ds(c0, CHUNK)]], rows_i, sem)
        g2 = pltpu.async_copy(pk_j_hbm.at[qidx_j.at[pl.ds(c0, CHUNK)]], rows_j, sem)
        g3 = pltpu.async_copy(bb_hbm.at[bidx_i.at[pl.ds(c0, CHUNK)]], brow_i, sem)
        g4 = pltpu.async_copy(bb_hbm.at[bidx_j.at[pl.ds(c0, CHUNK)]], brow_j, sem)
        g1.wait()
        g2.wait()
        g3.wait()
        g4.wait()

        def block(b, carry2):
            g0 = c0 + b * L
            s = pl.ds(g0, L)
            lrvec = b * L + iota
            wv_i = widx_i[s]
            wv_j = widx_j[s]
            col_i = (wv_i & 3) * PW
            col_j = (wv_j & 3) * PW
            acc = plsc.load_gather(brow_i, [lrvec, wv_i & 127]) + \
                plsc.load_gather(brow_j, [lrvec, (wv_j + V) & 127])
            for p in range(PW):
                pi = plsc.load_gather(rows_i, [lrvec, col_i + p])
                pj = plsc.load_gather(rows_j, [lrvec, col_j + p])
                fi_lo = plsc.bitcast(pi << 16, jnp.float32)
                fj_lo = plsc.bitcast(pj << 16, jnp.float32)
                fi_hi = plsc.bitcast(pi & lo_mask, jnp.float32)
                fj_hi = plsc.bitcast(pj & lo_mask, jnp.float32)
                acc = acc + fi_lo * fj_lo + fi_hi * fj_hi
            out_v[s] = acc
            return carry2

        lax.fori_loop(0, NBLK, block, 0, unroll=False)
        return carry

    lax.fori_loop(0, NCHUNK, chunk, 0, unroll=False)

    pltpu.sync_copy(out_v, out_hbm.at[pl.ds(base, BPW)])


@functools.partial(jax.jit, static_argnames=())
def kernel(word_i, word_j, wi, wj, bi, bj):
    mesh = plsc.VectorSubcoreMesh(core_axis_name="c", subcore_axis_name="s")
    k = pl.kernel(
        _body,
        out_type=jax.ShapeDtypeStruct((B,), jnp.float32),
        mesh=mesh,
        compiler_params=pltpu.CompilerParams(needs_layout_passes=False),
        scratch_types=[
            pltpu.VMEM((BPW,), jnp.int32),
            pltpu.VMEM((BPW,), jnp.int32),
            pltpu.VMEM((BPW,), jnp.int32),
            pltpu.VMEM((BPW,), jnp.int32),
            pltpu.VMEM((BPW,), jnp.int32),
            pltpu.VMEM((BPW,), jnp.int32),
            pltpu.VMEM((CHUNK, 128), jnp.int32),
            pltpu.VMEM((CHUNK, 128), jnp.int32),
            pltpu.VMEM((CHUNK, 128), jnp.float32),
            pltpu.VMEM((CHUNK, 128), jnp.float32),
            pltpu.VMEM((BPW,), jnp.float32),
            pltpu.SemaphoreType.DMA,
        ],
    )
    word_i = word_i.astype(jnp.int32)
    word_j = word_j.astype(jnp.int32)

    def rne16(u):
        return (u + 0x7FFF + ((u >> 16) & 1)) >> 16

    def pack(w):
        u = lax.bitcast_convert_type(w, jnp.int32)
        lo = rne16(u[:, : D // 2]) & 0xFFFF
        hi = rne16(u[:, D // 2:]) << 16
        return (lo | hi).reshape(V // 4, 128)

    pk_i = pack(wi)
    pk_j = pack(wj)
    bb = jnp.concatenate([bi.reshape(V), bj.reshape(V)]).reshape(2 * V // 128, 128)
    return k(word_i, word_j, pk_i, pk_j, bb)


# speedup vs baseline: 39.6233x; 1.3248x over previous
import functools

import jax
import jax.numpy as jnp
from jax import lax
from jax.experimental import pallas as pl
from jax.experimental.pallas import tpu as pltpu
from jax.experimental.pallas import tpu_sc as plsc

V = 1000000
D = 64
B = 16384

NC, NS, L = 2, 16, 16
NW = NC * NS
BPW = B // NW
NBLK = BPW // L


def _body(wi_i_hbm, wi_j_hbm, wi_hbm, wj_hbm, bi_hbm, bj_hbm, out_hbm,
          idx_i, idx_j, rows_i, rows_j, bv_i, bv_j, out_v, sem):
    wid = lax.axis_index("s") * NC + lax.axis_index("c")
    base = wid * BPW

    pltpu.sync_copy(wi_i_hbm.at[pl.ds(base, BPW)], idx_i)
    pltpu.sync_copy(wi_j_hbm.at[pl.ds(base, BPW)], idx_j)

    c1 = pltpu.async_copy(wi_hbm.at[idx_i], rows_i, sem)
    c2 = pltpu.async_copy(wj_hbm.at[idx_j], rows_j, sem)
    c3 = pltpu.async_copy(bi_hbm.at[idx_i], bv_i, sem)
    c4 = pltpu.async_copy(bj_hbm.at[idx_j], bv_j, sem)
    c1.wait()
    c2.wait()
    c3.wait()
    c4.wait()

    iota = lax.iota(jnp.int32, L)

    def block(b, carry):
        r0 = b * L
        acc = jnp.zeros((L,), jnp.float32)
        for r in range(L):
            row = r0 + r
            ri = rows_i.at[row]
            rj = rows_j.at[row]
            s = ri[pl.ds(0, L)] * rj[pl.ds(0, L)]
            for c in range(1, D // L):
                s = s + ri[pl.ds(c * L, L)] * rj[pl.ds(c * L, L)]
            acc = jnp.where(iota == r, jnp.sum(s), acc)
        acc = acc + bv_i[pl.ds(r0, L)] + bv_j[pl.ds(r0, L)]
        out_v[pl.ds(r0, L)] = acc
        return carry

    lax.fori_loop(0, NBLK, block, 0, unroll=False)

    pltpu.sync_copy(out_v, out_hbm.at[pl.ds(base, BPW)])


@functools.partial(jax.jit, static_argnames=())
def kernel(word_i, word_j, wi, wj, bi, bj):
    mesh = plsc.VectorSubcoreMesh(core_axis_name="c", subcore_axis_name="s")
    k = pl.kernel(
        _body,
        out_type=jax.ShapeDtypeStruct((B,), jnp.float32),
        mesh=mesh,
        compiler_params=pltpu.CompilerParams(
            needs_layout_passes=False, use_tc_tiling_on_sc=False),
        scratch_types=[
            pltpu.VMEM((BPW,), jnp.int32),
            pltpu.VMEM((BPW,), jnp.int32),
            pltpu.VMEM((BPW, D), jnp.float32),
            pltpu.VMEM((BPW, D), jnp.float32),
            pltpu.VMEM((BPW,), jnp.float32),
            pltpu.VMEM((BPW,), jnp.float32),
            pltpu.VMEM((BPW,), jnp.float32),
            pltpu.SemaphoreType.DMA,
        ],
    )
    return k(word_i.astype(jnp.int32), word_j.astype(jnp.int32), wi, wj,
             bi.reshape(V), bj.reshape(V))
